# baseline (device time: 106773 ns/iter reference)
import os

import jax
import jax.numpy as jnp
from jax import lax
from jax.experimental import pallas as pl
from jax.experimental.pallas import tpu as pltpu

_N_HOPS = int(os.environ.get("SK_HOPS", "3"))
_NO_EXP = os.environ.get("SK_NO_EXP") == "1"

N_DEV = 4
SCALE = 0.08838834764831843
BLK = 64


def kernel(x, Wq, K_ext, V_ext, Wo):
    B, Sq, D = x.shape
    _, Skv_l, Hq, Dh = K_ext.shape

    x2 = x.reshape(Sq, D).astype(jnp.bfloat16)
    wq = Wq.astype(jnp.bfloat16)
    k2 = K_ext.reshape(Skv_l, Hq * Dh)
    v2 = V_ext.reshape(Skv_l, Hq * Dh)
    wo = Wo.reshape(Hq, Dh, D).astype(jnp.bfloat16)

    def body(x_ref, wq_ref, k_ref, v_ref, wo_ref, out_ref,
             q_buf, o_comm, s_comm, acc_o, acc_s,
             o_send_sems, o_recv_sems, s_send_sems, s_recv_sems):
        my = lax.axis_index("i")
        left = lax.rem(my + N_DEV - 1, N_DEV)
        right = lax.rem(my + 1, N_DEV)

        barrier_sem = pltpu.get_barrier_semaphore()
        for nbr in (left, right):
            pl.semaphore_signal(barrier_sem, inc=1, device_id=(nbr,),
                                device_id_type=pl.DeviceIdType.MESH)
        pl.semaphore_wait(barrier_sem, 2)

        q = jnp.dot(x_ref[...], wq_ref[...],
                    preferred_element_type=jnp.float32) * SCALE
        for h in range(Hq):
            q_buf[h] = q[:, h * Dh:(h + 1) * Dh].astype(jnp.bfloat16)

        q_blk = lax.broadcasted_iota(jnp.int32, (Skv_l, Sq), 1) // BLK
        k_blk = (lax.broadcasted_iota(jnp.int32, (Skv_l, Sq), 0)
                 + my * Skv_l) // BLK
        mask = ((q_blk == k_blk) | (k_blk == 0)
                | (lax.rem(q_blk + k_blk, 3) == 0))
        bias = jnp.where(mask, 0.0, -1e9)

        def head_body(h, _):
            qh = q_buf[h]
            kh = k_ref[:, pl.ds(h * Dh, Dh)].astype(jnp.bfloat16)
            vh = v_ref[:, pl.ds(h * Dh, Dh)].astype(jnp.bfloat16)
            s = lax.dot_general(kh, qh,
                                (((1,), (1,)), ((), ())),
                                preferred_element_type=jnp.float32)
            if _NO_EXP:
                w = s
                m = jnp.max(s, axis=0, keepdims=True)
                lsum = jnp.sum(w, axis=0, keepdims=True)
            else:
                s = s + bias
                m = jnp.max(s, axis=0, keepdims=True)
                w = jnp.exp(s - m)
                lsum = jnp.sum(w, axis=0, keepdims=True)
            o = lax.dot_general(vh, w.astype(jnp.bfloat16),
                                (((0,), (0,)), ((), ())),
                                preferred_element_type=jnp.float32)
            acc_o[h] = o
            o_comm[0, h] = o.astype(jnp.bfloat16)
            acc_s[h, 0:1, :] = m
            acc_s[h, 1:2, :] = lsum
            s_comm[0, h, 0:1, :] = m
            s_comm[0, h, 1:2, :] = lsum
            return 0

        lax.fori_loop(0, Hq, head_body, 0)

        for hop in range(_N_HOPS):
            send_slot = hop % 2
            recv_slot = (hop + 1) % 2
            rdma_o = pltpu.make_async_remote_copy(
                src_ref=o_comm.at[send_slot],
                dst_ref=o_comm.at[recv_slot],
                send_sem=o_send_sems.at[send_slot],
                recv_sem=o_recv_sems.at[recv_slot],
                device_id=(right,),
                device_id_type=pl.DeviceIdType.MESH,
            )
            rdma_s = pltpu.make_async_remote_copy(
                src_ref=s_comm.at[send_slot],
                dst_ref=s_comm.at[recv_slot],
                send_sem=s_send_sems.at[send_slot],
                recv_sem=s_recv_sems.at[recv_slot],
                device_id=(right,),
                device_id_type=pl.DeviceIdType.MESH,
            )
            rdma_o.start()
            rdma_s.start()
            rdma_o.wait()
            rdma_s.wait()

            def comb_body(h, _):
                m_a = acc_s[h, 0:1, :]
                l_a = acc_s[h, 1:2, :]
                m_r = s_comm[recv_slot, h, 0:1, :]
                l_r = s_comm[recv_slot, h, 1:2, :]
                m_new = jnp.maximum(m_a, m_r)
                a = jnp.exp(m_a - m_new)
                b = jnp.exp(m_r - m_new)
                acc_s[h, 0:1, :] = m_new
                acc_s[h, 1:2, :] = l_a * a + l_r * b
                acc_o[h] = (acc_o[h] * a
                            + o_comm[recv_slot, h].astype(jnp.float32) * b)
                return 0

            lax.fori_loop(0, Hq, comb_body, 0)

        out_ref[...] = jnp.zeros((Sq, D), jnp.float32)

        def proj_body(h, _):
            ctx_h = (acc_o[h] / acc_s[h, 1:2, :]).astype(jnp.bfloat16)
            out_ref[...] += lax.dot_general(
                ctx_h, wo_ref[h], (((0,), (0,)), ((), ())),
                preferred_element_type=jnp.float32)
            return 0

        lax.fori_loop(0, Hq, proj_body, 0)

    out = pl.pallas_call(
        body,
        out_shape=jax.ShapeDtypeStruct((Sq, D), jnp.float32),
        in_specs=[pl.BlockSpec(memory_space=pltpu.VMEM)] * 5,
        out_specs=pl.BlockSpec(memory_space=pltpu.VMEM),
        scratch_shapes=[
            pltpu.VMEM((Hq, Sq, Dh), jnp.bfloat16),
            pltpu.VMEM((2, Hq, Dh, Sq), jnp.bfloat16),
            pltpu.VMEM((2, Hq, 2, Sq), jnp.float32),
            pltpu.VMEM((Hq, Dh, Sq), jnp.float32),
            pltpu.VMEM((Hq, 2, Sq), jnp.float32),
            pltpu.SemaphoreType.DMA((2,)),
            pltpu.SemaphoreType.DMA((2,)),
            pltpu.SemaphoreType.DMA((2,)),
            pltpu.SemaphoreType.DMA((2,)),
        ],
        compiler_params=pltpu.CompilerParams(
            collective_id=0,
            vmem_limit_bytes=128 * 1024 * 1024,
        ),
    )(x2, wq, k2, v2, wo)
    return out.reshape(B, Sq, D)


# device time: 69182 ns/iter; 1.5434x vs baseline; 1.5434x over previous
import jax
import jax.numpy as jnp
from jax import lax
from jax.experimental import pallas as pl
from jax.experimental.pallas import tpu as pltpu

N_DEV = 4
SCALE = 0.08838834764831843
BLK = 64


def kernel(x, Wq, K_ext, V_ext, Wo):
    B, Sq, D = x.shape
    _, Skv_l, Hq, Dh = K_ext.shape
    Hh = Hq // 2

    x2 = x.reshape(Sq, D).astype(jnp.bfloat16)
    wq = Wq.astype(jnp.bfloat16)
    k2 = K_ext.reshape(Skv_l, Hq * Dh).astype(jnp.bfloat16)
    v2 = V_ext.reshape(Skv_l, Hq * Dh).astype(jnp.bfloat16)
    wo = Wo.reshape(Hq, Dh, D).astype(jnp.bfloat16)

    def body(x_ref, wq_ref, k_hbm, v_hbm, wo_ref, out_ref,
             q_buf, k_ref, v_ref,
             o_loc, s_loc, r1_o, r1_s, c1_o, c1_s, r2_o, r2_s,
             acc_o, acc_s, kv_sems,
             o1_send, o1_recv, s1_send, s1_recv,
             o2_send, o2_recv, s2_send, s2_recv):
        my = lax.axis_index("i")
        p1 = my + 1 - 2 * lax.rem(my, 2)
        p2 = 3 - my

        k_dma = pltpu.make_async_copy(k_hbm, k_ref, kv_sems.at[0])
        v_dma = pltpu.make_async_copy(v_hbm, v_ref, kv_sems.at[1])
        k_dma.start()
        v_dma.start()

        barrier_sem = pltpu.get_barrier_semaphore()
        for nbr in (p1, p2):
            pl.semaphore_signal(barrier_sem, inc=1, device_id=(nbr,),
                                device_id_type=pl.DeviceIdType.MESH)

        q = jnp.dot(x_ref[...], wq_ref[...],
                    preferred_element_type=jnp.float32) * SCALE
        for h in range(Hq):
            q_buf[h] = q[:, h * Dh:(h + 1) * Dh].astype(jnp.bfloat16)

        q_blk = lax.broadcasted_iota(jnp.int32, (Skv_l, Sq), 1) // BLK
        k_blk = (lax.broadcasted_iota(jnp.int32, (Skv_l, Sq), 0)
                 + my * Skv_l) // BLK
        mask = ((q_blk == k_blk) | (k_blk == 0)
                | (lax.rem(q_blk + k_blk, 3) == 0))
        bias = jnp.where(mask, 0.0, -1e9)

        k_dma.wait()
        v_dma.wait()

        ones_row = jnp.ones((1, Skv_l), jnp.bfloat16)

        def head_body(h, _):
            qh = q_buf[h]
            kh = k_ref[:, pl.ds(h * Dh, Dh)]
            vh = v_ref[:, pl.ds(h * Dh, Dh)]
            s = lax.dot_general(kh, qh,
                                (((1,), (1,)), ((), ())),
                                preferred_element_type=jnp.float32)
            s = s + bias
            m = jnp.max(s, axis=0, keepdims=True)
            w = jnp.exp(s).astype(jnp.bfloat16)
            em = jnp.exp(-m)
            l_raw = lax.dot_general(ones_row, w,
                                    (((1,), (0,)), ((), ())),
                                    preferred_element_type=jnp.float32)
            o_raw = lax.dot_general(vh, w,
                                    (((0,), (0,)), ((), ())),
                                    preferred_element_type=jnp.float32)
            o = o_raw * em
            lsum = l_raw * em
            acc_o[h] = o
            o_loc[h] = o.astype(jnp.bfloat16)
            acc_s[h, 0:1, :] = m
            acc_s[h, 1:2, :] = lsum
            s_loc[h, 0:1, :] = m
            s_loc[h, 1:2, :] = lsum
            return 0

        def exchange(half, src_o, src_s, dst_o, dst_s, osend, orecv,
                     ssend, srecv, partner):
            sl = slice(half * Hh, (half + 1) * Hh)
            ro = pltpu.make_async_remote_copy(
                src_ref=src_o.at[sl], dst_ref=dst_o.at[sl],
                send_sem=osend.at[half], recv_sem=orecv.at[half],
                device_id=(partner,),
                device_id_type=pl.DeviceIdType.MESH)
            rs = pltpu.make_async_remote_copy(
                src_ref=src_s.at[sl], dst_ref=dst_s.at[sl],
                send_sem=ssend.at[half], recv_sem=srecv.at[half],
                device_id=(partner,),
                device_id_type=pl.DeviceIdType.MESH)
            ro.start()
            rs.start()
            return ro, rs

        def merge_half(half, r_o, r_s, write_c1):
            def cb(h, _):
                m_a = acc_s[h, 0:1, :]
                l_a = acc_s[h, 1:2, :]
                m_r = r_s[h, 0:1, :]
                l_r = r_s[h, 1:2, :]
                m_n = jnp.maximum(m_a, m_r)
                a = jnp.exp(m_a - m_n)
                b = jnp.exp(m_r - m_n)
                l_n = l_a * a + l_r * b
                o_n = acc_o[h] * a + r_o[h].astype(jnp.float32) * b
                acc_s[h, 0:1, :] = m_n
                acc_s[h, 1:2, :] = l_n
                acc_o[h] = o_n
                if write_c1:
                    c1_s[h, 0:1, :] = m_n
                    c1_s[h, 1:2, :] = l_n
                    c1_o[h] = o_n.astype(jnp.bfloat16)
                return 0

            lax.fori_loop(half * Hh, (half + 1) * Hh, cb, 0)

        lax.fori_loop(0, Hh, head_body, 0)
        pl.semaphore_wait(barrier_sem, 2)
        r1a = exchange(0, o_loc, s_loc, r1_o, r1_s,
                       o1_send, o1_recv, s1_send, s1_recv, p1)
        lax.fori_loop(Hh, Hq, head_body, 0)
        r1b = exchange(1, o_loc, s_loc, r1_o, r1_s,
                       o1_send, o1_recv, s1_send, s1_recv, p1)

        r1a[0].wait_recv()
        r1a[1].wait_recv()
        merge_half(0, r1_o, r1_s, True)
        r2a = exchange(0, c1_o, c1_s, r2_o, r2_s,
                       o2_send, o2_recv, s2_send, s2_recv, p2)
        r1b[0].wait_recv()
        r1b[1].wait_recv()
        merge_half(1, r1_o, r1_s, True)
        r2b = exchange(1, c1_o, c1_s, r2_o, r2_s,
                       o2_send, o2_recv, s2_send, s2_recv, p2)

        out_ref[...] = jnp.zeros((Sq, D), jnp.float32)

        def proj_body(h, _):
            ctx_h = (acc_o[h] / acc_s[h, 1:2, :]).astype(jnp.bfloat16)
            out_ref[...] += lax.dot_general(
                ctx_h, wo_ref[h], (((0,), (0,)), ((), ())),
                preferred_element_type=jnp.float32)
            return 0

        r2a[0].wait_recv()
        r2a[1].wait_recv()
        merge_half(0, r2_o, r2_s, False)
        lax.fori_loop(0, Hh, proj_body, 0)
        r2b[0].wait_recv()
        r2b[1].wait_recv()
        merge_half(1, r2_o, r2_s, False)
        lax.fori_loop(Hh, Hq, proj_body, 0)

        for r in (*r1a, *r1b, *r2a, *r2b):
            r.wait_send()

    out = pl.pallas_call(
        body,
        out_shape=jax.ShapeDtypeStruct((Sq, D), jnp.float32),
        in_specs=[
            pl.BlockSpec(memory_space=pltpu.VMEM),
            pl.BlockSpec(memory_space=pltpu.VMEM),
            pl.BlockSpec(memory_space=pl.ANY),
            pl.BlockSpec(memory_space=pl.ANY),
            pl.BlockSpec(memory_space=pltpu.VMEM),
        ],
        out_specs=pl.BlockSpec(memory_space=pltpu.VMEM),
        scratch_shapes=[
            pltpu.VMEM((Hq, Sq, Dh), jnp.bfloat16),
            pltpu.VMEM((Skv_l, Hq * Dh), jnp.bfloat16),
            pltpu.VMEM((Skv_l, Hq * Dh), jnp.bfloat16),
            pltpu.VMEM((Hq, Dh, Sq), jnp.bfloat16),
            pltpu.VMEM((Hq, 2, Sq), jnp.float32),
            pltpu.VMEM((Hq, Dh, Sq), jnp.bfloat16),
            pltpu.VMEM((Hq, 2, Sq), jnp.float32),
            pltpu.VMEM((Hq, Dh, Sq), jnp.bfloat16),
            pltpu.VMEM((Hq, 2, Sq), jnp.float32),
            pltpu.VMEM((Hq, Dh, Sq), jnp.bfloat16),
            pltpu.VMEM((Hq, 2, Sq), jnp.float32),
            pltpu.VMEM((Hq, Dh, Sq), jnp.float32),
            pltpu.VMEM((Hq, 2, Sq), jnp.float32),
            pltpu.SemaphoreType.DMA((2,)),
            pltpu.SemaphoreType.DMA((2,)),
            pltpu.SemaphoreType.DMA((2,)),
            pltpu.SemaphoreType.DMA((2,)),
            pltpu.SemaphoreType.DMA((2,)),
            pltpu.SemaphoreType.DMA((2,)),
            pltpu.SemaphoreType.DMA((2,)),
            pltpu.SemaphoreType.DMA((2,)),
            pltpu.SemaphoreType.DMA((2,)),
        ],
        compiler_params=pltpu.CompilerParams(
            collective_id=0,
            vmem_limit_bytes=128 * 1024 * 1024,
        ),
    )(x2, wq, k2, v2, wo)
    return out.reshape(B, Sq, D)
